# baseline (speedup 1.0000x reference)
import functools

import jax
import jax.numpy as jnp
from jax import lax
from jax.experimental import pallas as pl
from jax.experimental.pallas import tpu as pltpu
from jax.experimental.pallas import tpu_sc as plsc

_B, _T, _D = 4, 8192, 1024
_NC, _NS = 2, 16
_NW = _NC * _NS
_ROWS = _T // _NW
_CH = 32
_NCH = _ROWS // _CH

_mesh = plsc.VectorSubcoreMesh(core_axis_name="c", subcore_axis_name="s")


@functools.partial(
    pl.kernel,
    mesh=_mesh,
    out_type=jax.ShapeDtypeStruct((_B, _T, _D), jnp.float32),
    scratch_types=[
        pltpu.VMEM((_CH, _D), jnp.float32),
        pltpu.VMEM((_CH, _D), jnp.float32),
        pltpu.VMEM((_CH,), jnp.int32),
        pltpu.VMEM((_CH,), jnp.int32),
        pltpu.SemaphoreType.DMA,
        pltpu.SemaphoreType.DMA,
        pltpu.SemaphoreType.DMA,
        pltpu.SemaphoreType.DMA,
    ],
)
def _pe_broadcast(pe_hbm, out_hbm, buf0, buf1, idx0, idx1, rsem0, rsem1,
                  wsem0, wsem1):
    wid = lax.axis_index("s") * _NC + lax.axis_index("c")
    base = wid * _ROWS
    bufs = (buf0, buf1)
    idxs = (idx0, idx1)
    rsems = (rsem0, rsem1)
    wsems = (wsem0, wsem1)
    lane = lax.iota(jnp.int32, 16)

    def start_gather(buf_id, start):
        for k in range(_CH // 16):
            idxs[buf_id][pl.ds(k * 16, 16)] = start + k * 16 + lane
        return pltpu.async_copy(pe_hbm.at[idxs[buf_id]], bufs[buf_id],
                                rsems[buf_id])

    reads = [None, None]
    writes = [None, None]
    reads[0] = start_gather(0, base)
    for c in range(_NCH):
        i = c % 2
        j = (c + 1) % 2
        start = base + c * _CH
        reads[i].wait()
        writes[i] = [
            pltpu.async_copy(bufs[i], out_hbm.at[b, pl.ds(start, _CH)], wsems[i])
            for b in range(_B)
        ]
        if c + 1 < _NCH:
            if writes[j] is not None:
                for w in writes[j]:
                    w.wait()
                writes[j] = None
            reads[j] = start_gather(j, start + _CH)
    for ws in writes:
        if ws is not None:
            for w in ws:
                w.wait()


def kernel(x, pe_weight):
    del x
    return _pe_broadcast(pe_weight)


# speedup vs baseline: 1.0200x; 1.0200x over previous
import functools

import jax
import jax.numpy as jnp
from jax import lax
from jax.experimental import pallas as pl
from jax.experimental.pallas import tpu as pltpu
from jax.experimental.pallas import tpu_sc as plsc

_B, _T, _D = 4, 8192, 1024
_NC, _NS = 2, 16
_NW = _NC * _NS
_ROWS = _T // _NW
_CH = 48
_CHUNKS = [48, 48, 48, 48, 48, 16]
_NCH = len(_CHUNKS)

_mesh = plsc.VectorSubcoreMesh(core_axis_name="c", subcore_axis_name="s")


@functools.partial(
    pl.kernel,
    mesh=_mesh,
    out_type=jax.ShapeDtypeStruct((_B, _T, _D), jnp.float32),
    scratch_types=[
        pltpu.VMEM((_CH, _D), jnp.float32),
        pltpu.VMEM((_CH, _D), jnp.float32),
        pltpu.VMEM((_CH,), jnp.int32),
        pltpu.VMEM((_CH,), jnp.int32),
        pltpu.SemaphoreType.DMA,
        pltpu.SemaphoreType.DMA,
        pltpu.SemaphoreType.DMA,
        pltpu.SemaphoreType.DMA,
    ],
)
def _pe_broadcast(pe_hbm, out_hbm, buf0, buf1, idx0, idx1, rsem0, rsem1,
                  wsem0, wsem1):
    wid = lax.axis_index("s") * _NC + lax.axis_index("c")
    base = wid * _ROWS
    bufs = (buf0, buf1)
    idxs = (idx0, idx1)
    rsems = (rsem0, rsem1)
    wsems = (wsem0, wsem1)
    lane = lax.iota(jnp.int32, 16)
    offs = [sum(_CHUNKS[:c]) for c in range(_NCH)]

    def start_gather(buf_id, start, size):
        for k in range(size // 16):
            idxs[buf_id][pl.ds(k * 16, 16)] = start + k * 16 + lane
        return pltpu.async_copy(pe_hbm.at[idxs[buf_id].at[pl.ds(0, size)]],
                                bufs[buf_id].at[pl.ds(0, size)],
                                rsems[buf_id])

    reads = [None, None]
    writes = [None, None]
    reads[0] = start_gather(0, base, _CHUNKS[0])
    for c in range(_NCH):
        i = c % 2
        j = (c + 1) % 2
        start = base + offs[c]
        size = _CHUNKS[c]
        reads[i].wait()
        writes[i] = [
            pltpu.async_copy(bufs[i].at[pl.ds(0, size)],
                             out_hbm.at[b, pl.ds(start, size)], wsems[i])
            for b in range(_B)
        ]
        if c + 1 < _NCH:
            if writes[j] is not None:
                for w in writes[j]:
                    w.wait()
                writes[j] = None
            reads[j] = start_gather(j, base + offs[c + 1], _CHUNKS[c + 1])
    for ws in writes:
        if ws is not None:
            for w in ws:
                w.wait()


def kernel(x, pe_weight):
    del x
    return _pe_broadcast(pe_weight)


# speedup vs baseline: 1.0269x; 1.0067x over previous
import functools

import jax
import jax.numpy as jnp
from jax import lax
from jax.experimental import pallas as pl
from jax.experimental.pallas import tpu as pltpu
from jax.experimental.pallas import tpu_sc as plsc

_B, _T, _D = 4, 8192, 1024
_NC, _NS = 2, 16
_NW = _NC * _NS
_ROWS = _T // _NW
_PLAN = [(0, 64, 0), (64, 48, 1), (112, 64, 0), (176, 48, 1), (224, 32, 0)]

_mesh = plsc.VectorSubcoreMesh(core_axis_name="c", subcore_axis_name="s")


@functools.partial(
    pl.kernel,
    mesh=_mesh,
    out_type=jax.ShapeDtypeStruct((_B, _T, _D), jnp.float32),
    scratch_types=[
        pltpu.VMEM((64, _D), jnp.float32),
        pltpu.VMEM((48, _D), jnp.float32),
        pltpu.SemaphoreType.DMA,
        pltpu.SemaphoreType.DMA,
        pltpu.SemaphoreType.DMA,
        pltpu.SemaphoreType.DMA,
    ],
)
def _pe_broadcast(pe_hbm, out_hbm, buf0, buf1, rsem0, rsem1, wsem0, wsem1):
    wid = lax.axis_index("s") * _NC + lax.axis_index("c")
    base = wid * _ROWS
    bufs = (buf0, buf1)
    rsems = (rsem0, rsem1)
    wsems = (wsem0, wsem1)

    def start_read(off, size, s):
        return pltpu.async_copy(pe_hbm.at[pl.ds(base + off, size)],
                                bufs[s].at[pl.ds(0, size)], rsems[s])

    reads = [None, None]
    writes = [None, None]
    reads[_PLAN[0][2]] = start_read(*_PLAN[0])
    for c, (off, size, s) in enumerate(_PLAN):
        reads[s].wait()
        writes_c = [
            pltpu.async_copy(bufs[s].at[pl.ds(0, size)],
                             out_hbm.at[b, pl.ds(base + off, size)], wsems[s])
            for b in range(_B)
        ]
        if c + 1 < len(_PLAN):
            s2 = _PLAN[c + 1][2]
            if writes[s2] is not None:
                for w in writes[s2]:
                    w.wait()
                writes[s2] = None
            reads[s2] = start_read(*_PLAN[c + 1])
        writes[s] = writes_c
    for ws in writes:
        if ws is not None:
            for w in ws:
                w.wait()


def kernel(x, pe_weight):
    del x
    return _pe_broadcast(pe_weight)
